# baseline (speedup 1.0000x reference)
import functools

import jax
import jax.numpy as jnp
from jax import lax
from jax.experimental import pallas as pl
from jax.experimental.pallas import tpu as pltpu
from jax.experimental.pallas import tpu_sc as plsc

BATCH = 16384
KNOW = 32
PACK = 4
NC = 2
NS = 16
NW = NC * NS
BPW = BATCH // NW
CHUNK = 128
NCHUNK = BPW // CHUNK


def _sigmoid(x):
    return 1.0 / (1.0 + jnp.exp(-x))


@functools.partial(
    pl.kernel,
    mesh=plsc.VectorSubcoreMesh(core_axis_name="c", subcore_axis_name="s"),
    out_type=jax.ShapeDtypeStruct((BATCH,), jnp.float32),
    compiler_params=pltpu.CompilerParams(
        needs_layout_passes=False, use_tc_tiling_on_sc=False),
    scratch_types=[
        pltpu.VMEM((NCHUNK, CHUNK), jnp.int32),
        pltpu.VMEM((NCHUNK, CHUNK), jnp.int32),
        pltpu.VMEM((NCHUNK, CHUNK), jnp.int32),
        pltpu.VMEM((NCHUNK, CHUNK), jnp.int32),
        pltpu.VMEM((CHUNK, 128), jnp.float32),
        pltpu.VMEM((CHUNK, 128), jnp.float32),
        pltpu.VMEM((CHUNK, 128), jnp.float32),
        pltpu.VMEM((CHUNK, 128), jnp.float32),
        pltpu.VMEM((BPW // PACK, 128), jnp.float32),
        pltpu.VMEM((BPW,), jnp.float32),
        pltpu.VMEM((BPW,), jnp.float32),
        pltpu.SemaphoreType.DMA,
    ],
)
def _sc_compute_if(sid_hbm, q_hbm, qline_hbm, stud_hbm, diff_hbm, disc_hbm,
                   out_hbm, sid_v, qid_v, srow_v, qrow_v, pa, pb, da, db,
                   qline_v, disc_v, out_v, sem):
    wid = lax.axis_index("s") * NC + lax.axis_index("c")
    base = wid * BPW

    pltpu.sync_copy(sid_hbm.at[pl.ds(wid * NCHUNK, NCHUNK)], sid_v)
    pltpu.sync_copy(q_hbm.at[pl.ds(wid * NCHUNK, NCHUNK)], qid_v)
    qcp = pltpu.make_async_copy(
        qline_hbm.at[pl.ds(wid * (BPW // PACK), BPW // PACK)], qline_v, sem)
    qcp.start()

    for c in range(NCHUNK):
        for m in range(CHUNK // 16):
            sl = pl.ds(m * 16, 16)
            srow_v[c, sl] = sid_v[c, sl] >> 2
            qrow_v[c, sl] = qid_v[c, sl] >> 2

    pbufs = (pa, pb)
    dbufs = (da, db)

    def fire(h):
        cp_p = pltpu.make_async_copy(
            stud_hbm.at[srow_v.at[h]], pbufs[h % 2], sem)
        cp_d = pltpu.make_async_copy(
            diff_hbm.at[qrow_v.at[h]], dbufs[h % 2], sem)
        cp_c = pltpu.make_async_copy(
            disc_hbm.at[qid_v.at[h]], disc_v.at[pl.ds(h * CHUNK, CHUNK)], sem)
        for cp in (cp_p, cp_d, cp_c):
            cp.start()
        return (cp_p, cp_d, cp_c)

    lanes = lax.iota(jnp.int32, 16)
    pending = fire(0)
    qcp.wait()

    for h in range(NCHUNK):
        for cp in pending:
            cp.wait()
        if h + 1 < NCHUNK:
            pending = fire(h + 1)
        pv = pbufs[h % 2]
        dv = dbufs[h % 2]

        def block_body(b, _, h=h, pv=pv, dv=dv):
            rows = b * 16 + lanes
            sl = pl.ds(b * 16, 16)
            e = h * CHUNK + b * 16 + lanes
            sub_s = (sid_v[h, sl] & 3) * KNOW
            sub_q = (qid_v[h, sl] & 3) * KNOW
            qrows = e >> 2
            qcols = (e & 3) * KNOW
            acc = jnp.zeros((16,), jnp.float32)

            def t_body(t, acc):
                p = plsc.load_gather(pv, [rows, sub_s + t])
                d = plsc.load_gather(dv, [rows, sub_q + t])
                qv = plsc.load_gather(qline_v, [qrows, qcols + t])
                return acc + qv * (_sigmoid(p) - _sigmoid(d))

            acc = lax.fori_loop(0, KNOW, t_body, acc)
            osl = pl.ds(h * CHUNK + b * 16, 16)
            disc16 = disc_v[osl]
            out_v[osl] = _sigmoid(_sigmoid(disc16) * acc)
            return 0

        lax.fori_loop(0, CHUNK // 16, block_body, 0)

    pltpu.sync_copy(out_v, out_hbm.at[pl.ds(base, BPW)])


def kernel(student_id, question, q_matrix_line, student_emb_w, difficulty_w,
           discrimination_w):
    sid2 = student_id.astype(jnp.int32).reshape(BATCH // CHUNK, CHUNK)
    q2 = question.astype(jnp.int32).reshape(BATCH // CHUNK, CHUNK)
    stud4 = student_emb_w.reshape(-1, 128)
    diff4 = difficulty_w.reshape(-1, 128)
    q4 = q_matrix_line.reshape(-1, 128)
    return _sc_compute_if(sid2, q2, q4, stud4, diff4,
                          discrimination_w.reshape(-1))


# speedup vs baseline: 1.0384x; 1.0384x over previous
import functools

import jax
import jax.numpy as jnp
from jax import lax
from jax.experimental import pallas as pl
from jax.experimental.pallas import tpu as pltpu
from jax.experimental.pallas import tpu_sc as plsc

BATCH = 16384
KNOW = 32
NC = 2
NS = 16
NW = NC * NS
BPW = BATCH // NW
CHUNK = 128
NCHUNK = BPW // CHUNK


def _sigmoid(x):
    return 1.0 / (1.0 + jnp.exp(-x))


@functools.partial(
    pl.kernel,
    mesh=plsc.VectorSubcoreMesh(core_axis_name="c", subcore_axis_name="s"),
    out_type=jax.ShapeDtypeStruct((BATCH,), jnp.float32),
    compiler_params=pltpu.CompilerParams(
        needs_layout_passes=False, use_tc_tiling_on_sc=False),
    scratch_types=[
        pltpu.VMEM((NCHUNK, CHUNK), jnp.int32),
        pltpu.VMEM((NCHUNK, CHUNK), jnp.int32),
        pltpu.VMEM((NCHUNK, CHUNK, KNOW), jnp.float32),
        pltpu.VMEM((NCHUNK, CHUNK, KNOW), jnp.float32),
        pltpu.VMEM((NCHUNK, CHUNK, KNOW), jnp.float32),
        pltpu.VMEM((NCHUNK, CHUNK), jnp.float32),
        pltpu.VMEM((BPW,), jnp.float32),
        pltpu.SemaphoreType.DMA,
        pltpu.SemaphoreType.DMA,
        pltpu.SemaphoreType.DMA,
    ],
)
def _sc_compute_if(sid_hbm, q_hbm, qline_hbm, stud_hbm, diff_hbm, disc_hbm,
                   out_hbm, sid_v, qid_v, pr, dr, qr, disc_v, out_v,
                   sem_a, sem_b, sem_q):
    wid = lax.axis_index("s") * NC + lax.axis_index("c")
    base = wid * BPW

    for c in range(NCHUNK):
        pltpu.sync_copy(sid_hbm.at[pl.ds(base + c * CHUNK, CHUNK)],
                        sid_v.at[c])
        pltpu.sync_copy(q_hbm.at[pl.ds(base + c * CHUNK, CHUNK)],
                        qid_v.at[c])

    qcps = [
        pltpu.make_async_copy(
            qline_hbm.at[pl.ds(base + c * CHUNK, CHUNK)], qr.at[c], sem_q)
        for c in range(NCHUNK)
    ]
    for cp in qcps:
        cp.start()

    sems = (sem_a, sem_b)

    def fire(c):
        sem = sems[c % 2]
        cps = [
            pltpu.make_async_copy(stud_hbm.at[sid_v.at[c]], pr.at[c], sem),
            pltpu.make_async_copy(diff_hbm.at[qid_v.at[c]], dr.at[c], sem),
            pltpu.make_async_copy(disc_hbm.at[qid_v.at[c]], disc_v.at[c],
                                  sem),
        ]
        for cp in cps:
            cp.start()
        return cps

    pending = fire(0)
    nxt = fire(1)
    for cp in qcps:
        cp.wait()

    lanes = lax.iota(jnp.int32, 16)

    for c in range(NCHUNK):
        for cp in pending:
            cp.wait()
        pending = nxt
        if c + 2 < NCHUNK:
            nxt = fire(c + 2)

        def block_body(b, _, c=c):
            acc = jnp.zeros((16,), jnp.float32)
            for j in range(16):
                i = b * 16 + j
                p0 = pr[c, i, pl.ds(0, 16)]
                p1 = pr[c, i, pl.ds(16, 16)]
                d0 = dr[c, i, pl.ds(0, 16)]
                d1 = dr[c, i, pl.ds(16, 16)]
                q0 = qr[c, i, pl.ds(0, 16)]
                q1 = qr[c, i, pl.ds(16, 16)]
                f = (q0 * (_sigmoid(p0) - _sigmoid(d0))
                     + q1 * (_sigmoid(p1) - _sigmoid(d1)))
                acc = jnp.where(lanes == j, jnp.sum(f), acc)
            out = _sigmoid(_sigmoid(disc_v[c, pl.ds(b * 16, 16)]) * acc)
            out_v[pl.ds(c * CHUNK + b * 16, 16)] = out
            return 0

        lax.fori_loop(0, CHUNK // 16, block_body, 0)

    pltpu.sync_copy(out_v, out_hbm.at[pl.ds(base, BPW)])


def kernel(student_id, question, q_matrix_line, student_emb_w, difficulty_w,
           discrimination_w):
    return _sc_compute_if(student_id.astype(jnp.int32),
                          question.astype(jnp.int32), q_matrix_line,
                          student_emb_w, difficulty_w,
                          discrimination_w.reshape(-1))
